# Initial kernel scaffold; baseline (speedup 1.0000x reference)
import jax
import jax.numpy as jnp
from jax.experimental import pallas as pl


def kernel(x, edge_index, edge_weight, conv_w, conv_b, bn_gamma, bn_beta, W1, b1, W2, b2, Wc, bc):
    raise NotImplementedError("write your pallas kernel here")


# speedup vs baseline: 5.4506x; 5.4506x over previous
import functools

import jax
import jax.numpy as jnp
from jax import lax
from jax.experimental import pallas as pl
from jax.experimental.pallas import tpu as pltpu
from jax.experimental.pallas import tpu_sc as plsc

N_CORES = 2
N_SUBCORES = 16
LANES = 16
NW = N_CORES * N_SUBCORES

_SC_MESH = plsc.VectorSubcoreMesh(
    core_axis_name="c", subcore_axis_name="s",
    num_cores=N_CORES, num_subcores=N_SUBCORES)


def _dense1_body(x_ref, cw0_ref, cw1_ref, cw2_ref, cb_ref, g_ref, b_ref,
                 w1_ref, out_ref):
    x = x_ref[...]
    s = jnp.dot(x, cw0_ref[...], preferred_element_type=jnp.float32)
    t = jnp.dot(x, cw1_ref[...], preferred_element_type=jnp.float32)
    u = jnp.dot(x, cw2_ref[...], preferred_element_type=jnp.float32)
    zrow = jnp.zeros((1, s.shape[1]), jnp.float32)
    h1 = (t
          + jnp.concatenate([zrow, s[:-1]], axis=0)
          + jnp.concatenate([u[1:], zrow], axis=0)
          + cb_ref[...])
    mean = jnp.mean(h1, axis=0, keepdims=True)
    var = jnp.mean((h1 - mean) ** 2, axis=0, keepdims=True)
    hn = (h1 - mean) * lax.rsqrt(var + 1e-5) * g_ref[...] + b_ref[...]
    hn = jnp.maximum(hn, 0.0)
    out_ref[...] = jnp.dot(hn, w1_ref[...], preferred_element_type=jnp.float32)


def _dense1(x, cw0, cw1, cw2, cb, g, b, w1):
    n = x.shape[0]
    f = w1.shape[1]
    return pl.pallas_call(
        _dense1_body,
        out_shape=jax.ShapeDtypeStruct((n, f), jnp.float32),
    )(x, cw0, cw1, cw2, cb, g, b, w1)


def _dense2_body(acc_ref, b1_ref, w2_ref, feat2_ref, s2_ref):
    h2 = jnp.maximum(acc_ref[0] + acc_ref[1] + b1_ref[...], 0.0)
    feat2_ref[...] = jnp.dot(h2, w2_ref[...], preferred_element_type=jnp.float32)
    s2_ref[...] = jnp.sum(h2, axis=0, keepdims=True)


def _dense2(acc, b1, w2p):
    n = acc.shape[1]
    f1 = acc.shape[2]
    f2 = w2p.shape[1]
    return pl.pallas_call(
        _dense2_body,
        out_shape=(jax.ShapeDtypeStruct((n, f2), jnp.float32),
                   jax.ShapeDtypeStruct((1, f1), jnp.float32)),
    )(acc, b1, w2p)


def _dense3_body(acc_ref, b2_ref, s2_ref, wc1_ref, wc2_ref, bc_ref, out_ref):
    h3 = jnp.maximum(acc_ref[0] + acc_ref[1] + b2_ref[...], 0.0)
    s3 = jnp.sum(h3, axis=0, keepdims=True)
    out_ref[...] = (
        jnp.dot(s2_ref[...], wc1_ref[...], preferred_element_type=jnp.float32)
        + jnp.dot(s3[:, :wc2_ref.shape[0]], wc2_ref[...],
                  preferred_element_type=jnp.float32)
        + bc_ref[...])


def _dense3(acc, b2p, s2, wc1, wc2, bc):
    return pl.pallas_call(
        _dense3_body,
        out_shape=jax.ShapeDtypeStruct((1, wc1.shape[1]), jnp.float32),
    )(acc, b2p, s2, wc1, wc2, bc)


@functools.lru_cache(maxsize=None)
def _make_edge_pass(n, e, f, chunk):
    epw = e // NW
    chunks = epw // chunk

    def body(feat_hbm, src_hbm, dst_hbm, ew_hbm, zeros_hbm, out_hbm,
             src_v, dst_v, ew_v, rows_v, acc_sh):
        cid = lax.axis_index("c")
        sid = lax.axis_index("s")
        wid = sid * N_CORES + cid

        pltpu.sync_copy(src_hbm.at[wid], src_v)
        pltpu.sync_copy(dst_hbm.at[wid], dst_v)
        pltpu.sync_copy(ew_hbm.at[pl.ds(wid * epw, epw)], ew_v)

        @pl.when(sid == 0)
        def _():
            pltpu.sync_copy(zeros_hbm, acc_sh)

        plsc.subcore_barrier()

        def chunk_body(i, carry):
            pltpu.sync_copy(feat_hbm.at[src_v.at[i]], rows_v)

            def group_body(gi, c2):
                wv = ew_v[pl.ds(i * chunk + gi * LANES, LANES)]
                for l in range(LANES):
                    ei = gi * LANES + l
                    w = wv[l]
                    for j in range(f // LANES):
                        blk = rows_v[ei, pl.ds(j * LANES, LANES)]
                        rows_v[ei, pl.ds(j * LANES, LANES)] = blk * w
                return c2

            lax.fori_loop(0, chunk // LANES, group_body, 0)
            pltpu.sync_copy(rows_v, acc_sh.at[dst_v.at[i]], add=True)
            return carry

        lax.fori_loop(0, chunks, chunk_body, 0)
        plsc.subcore_barrier()

        @pl.when(sid == 0)
        def _():
            pltpu.sync_copy(acc_sh, out_hbm.at[cid])

    return pl.kernel(
        body,
        out_type=jax.ShapeDtypeStruct((N_CORES, n, f), jnp.float32),
        mesh=_SC_MESH,
        scratch_types=[
            pltpu.VMEM((chunks, chunk), jnp.int32),
            pltpu.VMEM((chunks, chunk), jnp.int32),
            pltpu.VMEM((epw,), jnp.float32),
            pltpu.VMEM((chunk, f), jnp.float32),
            pltpu.VMEM_SHARED((n, f), jnp.float32),
        ],
        compiler_params=pltpu.CompilerParams(use_tc_tiling_on_sc=False),
    )


def _edge_pass(feat, src2d, dst2d, ew, zeros, chunk):
    n, f = feat.shape
    e = ew.shape[0]
    return _make_edge_pass(n, e, f, chunk)(feat, src2d, dst2d, ew, zeros)


def kernel(x, edge_index, edge_weight, conv_w, conv_b, bn_gamma, bn_beta,
           W1, b1, W2, b2, Wc, bc):
    n = x.shape[0]
    e = edge_index.shape[1]
    f1 = W1.shape[1]
    f2p = 64

    chunk = 80
    cw0 = conv_w[:, :, 0].T
    cw1 = conv_w[:, :, 1].T
    cw2 = conv_w[:, :, 2].T
    feat1 = _dense1(x, cw0, cw1, cw2, conv_b[None], bn_gamma[None],
                    bn_beta[None], W1)

    src2d = edge_index[0].reshape(NW, e // (NW * chunk), chunk)
    dst2d = edge_index[1].reshape(NW, e // (NW * chunk), chunk)

    zeros1 = jnp.zeros((n, f1), jnp.float32)
    acc1 = _edge_pass(feat1, src2d, dst2d, edge_weight, zeros1, chunk)

    w2p = jnp.zeros((f1, f2p), jnp.float32).at[:, :W2.shape[1]].set(W2)
    feat2, s2 = _dense2(acc1, b1[None], w2p)

    zeros2 = jnp.zeros((n, f2p), jnp.float32)
    acc2 = _edge_pass(feat2, src2d, dst2d, edge_weight, zeros2, chunk)

    b2p = jnp.zeros((1, f2p), jnp.float32).at[0, :b2.shape[0]].set(b2)
    wc1 = Wc[:f1]
    wc2 = Wc[f1:]
    return _dense3(acc2, b2p, s2, wc1, wc2, bc)
